# Initial kernel scaffold; baseline (speedup 1.0000x reference)
import jax
import jax.numpy as jnp
from jax.experimental import pallas as pl


def kernel(x, batch):
    raise NotImplementedError("write your pallas kernel here")


# speedup vs baseline: 4.8087x; 4.8087x over previous
import functools

import jax
import jax.numpy as jnp
from jax import lax
from jax.experimental import pallas as pl
from jax.experimental.pallas import tpu as pltpu
from jax.experimental.pallas import tpu_sc as plsc

N = 100000
D = 128
S = 512
NC = 2
NS = 16
NW = NC * NS
RPW = N // NW
C = 125
NCH = RPW // C
ROWS_PER_TILE_OUT = S // NS


def _sc_body(x_hbm, b_hbm, out_hbm, idx_v, xbuf, zbuf, acc_sh):
    c = lax.axis_index("c")
    s = lax.axis_index("s")
    wid = s * NC + c

    zrow = jnp.zeros((16,), jnp.float32)

    def _zero_row(i, carry):
        for cc in range(D // 16):
            zbuf[i, pl.ds(cc * 16, 16)] = zrow
        return carry

    lax.fori_loop(0, ROWS_PER_TILE_OUT, _zero_row, 0)
    pltpu.sync_copy(zbuf, acc_sh.at[pl.ds(s * ROWS_PER_TILE_OUT,
                                          ROWS_PER_TILE_OUT)])

    pltpu.sync_copy(b_hbm.at[pl.ds(wid * NCH, NCH)], idx_v)
    plsc.subcore_barrier()

    base = wid * RPW

    def _chunk(j, carry):
        pltpu.sync_copy(x_hbm.at[pl.ds(base + j * C, C)], xbuf)
        pltpu.sync_copy(xbuf, acc_sh.at[idx_v.at[j]], add=True)
        return carry

    lax.fori_loop(0, NCH, _chunk, 0)
    plsc.subcore_barrier()

    pltpu.sync_copy(
        acc_sh.at[pl.ds(s * ROWS_PER_TILE_OUT, ROWS_PER_TILE_OUT)],
        out_hbm.at[c, pl.ds(s * ROWS_PER_TILE_OUT, ROWS_PER_TILE_OUT)])


_sc_call = functools.partial(
    pl.kernel,
    out_type=jax.ShapeDtypeStruct((NC, S, D), jnp.float32),
    mesh=plsc.VectorSubcoreMesh(core_axis_name="c", subcore_axis_name="s"),
    scratch_types=[
        pltpu.VMEM((NCH, C), jnp.int32),
        pltpu.VMEM((C, D), jnp.float32),
        pltpu.VMEM((ROWS_PER_TILE_OUT, D), jnp.float32),
        pltpu.VMEM_SHARED((S, D), jnp.float32),
    ],
    compiler_params=pltpu.CompilerParams(use_tc_tiling_on_sc=False),
)(_sc_body)


def _combine_body(p_ref, o_ref):
    o_ref[...] = p_ref[0] + p_ref[1]


def kernel(x, batch):
    b2 = batch.astype(jnp.int32).reshape(N // C, C)
    partials = _sc_call(x, b2)
    return pl.pallas_call(
        _combine_body,
        out_shape=jax.ShapeDtypeStruct((S, D), jnp.float32),
    )(partials)


# speedup vs baseline: 6.1909x; 1.2874x over previous
import functools

import jax
import jax.numpy as jnp
from jax import lax
from jax.experimental import pallas as pl
from jax.experimental.pallas import tpu as pltpu
from jax.experimental.pallas import tpu_sc as plsc

N = 100000
D = 128
S = 512
NC = 2
NS = 16
NW = NC * NS
RPW = N // NW
C = 125
NCH = RPW // C
ROWS_PER_TILE_OUT = S // NS


def _sc_body(x_hbm, b_hbm, out_hbm, idx_v, xbuf0, xbuf1, zbuf, acc_sh,
             sem0, sem1):
    c = lax.axis_index("c")
    s = lax.axis_index("s")
    wid = s * NC + c
    base = wid * RPW

    pltpu.async_copy(x_hbm.at[pl.ds(base, C)], xbuf0, sem0)

    zrow = jnp.zeros((16,), jnp.float32)

    def _zero_row(i, carry):
        for cc in range(D // 16):
            zbuf[i, pl.ds(cc * 16, 16)] = zrow
        return carry

    lax.fori_loop(0, ROWS_PER_TILE_OUT, _zero_row, 0)
    pltpu.sync_copy(zbuf, acc_sh.at[pl.ds(s * ROWS_PER_TILE_OUT,
                                          ROWS_PER_TILE_OUT)])

    pltpu.sync_copy(b_hbm.at[pl.ds(wid * NCH, NCH)], idx_v)
    plsc.subcore_barrier()

    def _pair(p, carry):
        j0 = p * 2
        pltpu.async_copy(x_hbm.at[pl.ds(base + (j0 + 1) * C, C)], xbuf1,
                         sem1)
        pltpu.make_async_copy(x_hbm.at[pl.ds(base, C)], xbuf0, sem0).wait()
        pltpu.sync_copy(xbuf0, acc_sh.at[idx_v.at[j0]], add=True)
        pltpu.async_copy(x_hbm.at[pl.ds(base + (j0 + 2) * C, C)], xbuf0,
                         sem0)
        pltpu.make_async_copy(x_hbm.at[pl.ds(base, C)], xbuf1, sem1).wait()
        pltpu.sync_copy(xbuf1, acc_sh.at[idx_v.at[j0 + 1]], add=True)
        return carry

    lax.fori_loop(0, (NCH - 1) // 2, _pair, 0)
    pltpu.make_async_copy(x_hbm.at[pl.ds(base, C)], xbuf0, sem0).wait()
    pltpu.sync_copy(xbuf0, acc_sh.at[idx_v.at[NCH - 1]], add=True)
    plsc.subcore_barrier()

    pltpu.sync_copy(
        acc_sh.at[pl.ds(s * ROWS_PER_TILE_OUT, ROWS_PER_TILE_OUT)],
        out_hbm.at[c, pl.ds(s * ROWS_PER_TILE_OUT, ROWS_PER_TILE_OUT)])


_sc_call = functools.partial(
    pl.kernel,
    out_type=jax.ShapeDtypeStruct((NC, S, D), jnp.float32),
    mesh=plsc.VectorSubcoreMesh(core_axis_name="c", subcore_axis_name="s"),
    scratch_types=[
        pltpu.VMEM((NCH, C), jnp.int32),
        pltpu.VMEM((C, D), jnp.float32),
        pltpu.VMEM((C, D), jnp.float32),
        pltpu.VMEM((ROWS_PER_TILE_OUT, D), jnp.float32),
        pltpu.VMEM_SHARED((S, D), jnp.float32),
        pltpu.SemaphoreType.DMA,
        pltpu.SemaphoreType.DMA,
    ],
    compiler_params=pltpu.CompilerParams(use_tc_tiling_on_sc=False),
)(_sc_body)


def _combine_body(p_ref, o_ref):
    o_ref[...] = p_ref[0] + p_ref[1]


def kernel(x, batch):
    b2 = batch.astype(jnp.int32).reshape(N // C, C)
    partials = _sc_call(x, b2)
    return pl.pallas_call(
        _combine_body,
        out_shape=jax.ShapeDtypeStruct((S, D), jnp.float32),
    )(partials)
